# baseline (speedup 1.0000x reference)
import functools

import jax
import jax.numpy as jnp
from jax import lax
from jax.experimental import pallas as pl
from jax.experimental.pallas import tpu as pltpu


def _ceil_div(a, b):
    return -(-a // b)


def _loss_tile_kernel(a_ref, p_ref, n_ref, o_ref, *,
                      margin, rows_total, tile_rows, inner, inv_batch,
                      need_mask):
    i = pl.program_id(1)

    @pl.when(i == 0)
    def _init():
        o_ref[0, 0, 0] = 0.0

    a = a_ref[...].astype(jnp.float32)
    p = p_ref[...].astype(jnp.float32)
    n = n_ref[...].astype(jnp.float32)

    dp = a - p
    dn = a - n
    diff = dp * dp - dn * dn
    d = jnp.sum(diff, axis=-1, keepdims=True)
    per_row = jnp.maximum(d + margin, 0.0)

    if need_mask:
        tile = pl.program_id(0) * inner + i
        rows = tile * tile_rows + lax.broadcasted_iota(
            jnp.int32, per_row.shape, 0)
        per_row = jnp.where(rows < rows_total, per_row, 0.0)

    o_ref[0, 0, 0] += jnp.sum(per_row) * inv_batch


def kernel(anchor, positive, negative, margin=1.0, tile_rows=None):
    assert anchor.shape == positive.shape == negative.shape
    feat = anchor.shape[-1]
    anchor = anchor.reshape(-1, feat)
    positive = positive.reshape(-1, feat)
    negative = negative.reshape(-1, feat)
    batch = anchor.shape[0]

    lane_cols = _ceil_div(feat, 128) * 128
    itemsize = jnp.dtype(anchor.dtype).itemsize
    if tile_rows is None:
        tile_rows = max(8, (2 * 1024 * 1024 // (lane_cols * itemsize))
                        // 8 * 8)
        if tile_rows >= batch:
            tile_rows = batch
    tile_rows = int(tile_rows)
    assert tile_rows == batch or tile_rows % 8 == 0

    num_tiles = _ceil_div(batch, tile_rows)
    outer = 2 if num_tiles >= 2 else 1
    inner = _ceil_div(num_tiles, outer)
    need_mask = (outer * inner * tile_rows != batch)

    if outer * inner == num_tiles:
        def row_block(o, i):
            return (o * inner + i, 0)
    else:
        def row_block(o, i):
            return (jnp.minimum(o * inner + i, num_tiles - 1), 0)

    kernel_fn = functools.partial(
        _loss_tile_kernel, margin=float(margin), rows_total=batch,
        tile_rows=tile_rows, inner=inner, inv_batch=1.0 / batch,
        need_mask=need_mask)

    in_spec = pl.BlockSpec((tile_rows, feat), row_block)

    partial = pl.pallas_call(
        kernel_fn,
        out_shape=jax.ShapeDtypeStruct((outer, 1, 1), jnp.float32),
        grid=(outer, inner),
        in_specs=[in_spec, in_spec, in_spec],
        out_specs=pl.BlockSpec((1, 1, 1), lambda o, i: (o, 0, 0),
                               memory_space=pltpu.SMEM),
        compiler_params=pltpu.CompilerParams(
            dimension_semantics=("parallel", "arbitrary"),
            vmem_limit_bytes=48 * 1024 * 1024),
    )(anchor, positive, negative)

    return jnp.sum(partial)


# speedup vs baseline: 1.0145x; 1.0145x over previous
import functools

import jax
import jax.numpy as jnp
from jax import lax
from jax.experimental import pallas as pl
from jax.experimental.pallas import tpu as pltpu


def _ceil_div(a, b):
    return -(-a // b)


def _loss_tile_kernel(a_ref, p_ref, n_ref, o_ref, *,
                      margin, rows_total, tile_rows, num_tiles, inv_batch,
                      need_mask):
    i = pl.program_id(0)

    @pl.when(i == 0)
    def _init():
        o_ref[0, 0] = 0.0

    a = a_ref[...].astype(jnp.float32)
    p = p_ref[...].astype(jnp.float32)
    n = n_ref[...].astype(jnp.float32)

    dp = a - p
    dn = a - n
    diff = dp * dp - dn * dn
    d = jnp.sum(diff, axis=-1, keepdims=True)
    per_row = jnp.maximum(d + margin, 0.0)

    if need_mask:
        rows = i * tile_rows + lax.broadcasted_iota(
            jnp.int32, per_row.shape, 0)
        per_row = jnp.where(rows < rows_total, per_row, 0.0)

    o_ref[0, 0] += jnp.sum(per_row) * inv_batch


def kernel(anchor, positive, negative, margin=1.0, tile_rows=None):
    assert anchor.shape == positive.shape == negative.shape
    feat = anchor.shape[-1]
    anchor = anchor.reshape(-1, feat)
    positive = positive.reshape(-1, feat)
    negative = negative.reshape(-1, feat)
    batch = anchor.shape[0]

    lane_cols = _ceil_div(feat, 128) * 128
    itemsize = jnp.dtype(anchor.dtype).itemsize
    if tile_rows is None:
        tile_rows = max(8, (2 * 1024 * 1024 // (lane_cols * itemsize))
                        // 8 * 8)
        if tile_rows >= batch:
            tile_rows = batch
    tile_rows = int(tile_rows)
    assert tile_rows == batch or tile_rows % 8 == 0

    num_tiles = _ceil_div(batch, tile_rows)
    need_mask = (num_tiles * tile_rows != batch)

    kernel_fn = functools.partial(
        _loss_tile_kernel, margin=float(margin), rows_total=batch,
        tile_rows=tile_rows, num_tiles=num_tiles, inv_batch=1.0 / batch,
        need_mask=need_mask)

    in_spec = pl.BlockSpec((tile_rows, feat), lambda i: (i, 0))

    out = pl.pallas_call(
        kernel_fn,
        out_shape=jax.ShapeDtypeStruct((1, 1), jnp.float32),
        grid=(num_tiles,),
        in_specs=[in_spec, in_spec, in_spec],
        out_specs=pl.BlockSpec(memory_space=pltpu.SMEM),
        compiler_params=pltpu.CompilerParams(
            dimension_semantics=("arbitrary",),
            vmem_limit_bytes=48 * 1024 * 1024),
    )(anchor, positive, negative)

    return out.reshape(())
